# baseline (speedup 1.0000x reference)
import functools

import jax
import jax.numpy as jnp
from jax import lax
from jax.experimental import pallas as pl
from jax.experimental.pallas import tpu as pltpu
from jax.experimental.pallas import tpu_sc as plsc

B, M, K, DSUB = 16384, 96, 256, 8
D = M * DSUB
HALVES = 2
MH = M // HALVES
DH = D // HALVES
TW = MH * K * DSUB
MW = K * DSUB
NSUB = 16
ROWS_PER_SUB = B // NSUB
CHUNK = 16
NCHUNK = ROWS_PER_SUB // CHUNK
NACC = 5
JV = DH // 16


def _sc_body(q_hbm, pos_hbm, neg_hbm, tab_hbm, out_hbm,
             tab_v, qb, pb, nb, ob, sem0, sem1):
    c = lax.axis_index("c")
    s = lax.axis_index("s")
    base_row = s * ROWS_PER_SUB

    pltpu.sync_copy(tab_hbm.at[c], tab_v)

    iota = lax.broadcasted_iota(jnp.int32, (16,), 0)
    hi = lax.shift_right_logical(iota, 3)
    d8 = lax.bitwise_and(iota, 7)
    tpat = hi * MW + d8 * K

    def fire(g, slot, sem, colq0):
        r0 = base_row + g * CHUNK
        dst = pl.ds(slot * CHUNK, CHUNK)
        pltpu.async_copy(q_hbm.at[pl.ds(r0, CHUNK), pl.ds(colq0, DH)],
                         qb.at[dst], sem)
        pltpu.async_copy(pos_hbm.at[pl.ds(r0, CHUNK)], pb.at[dst], sem)
        pltpu.async_copy(neg_hbm.at[pl.ds(r0, CHUNK)], nb.at[dst], sem)

    def wait(g, slot, sem, colq0):
        r0 = base_row + g * CHUNK
        dst = pl.ds(slot * CHUNK, CHUNK)
        pltpu.make_async_copy(
            q_hbm.at[pl.ds(r0, CHUNK), pl.ds(colq0, DH)],
            qb.at[dst], sem).wait()
        pltpu.make_async_copy(pos_hbm.at[pl.ds(r0, CHUNK)],
                              pb.at[dst], sem).wait()
        pltpu.make_async_copy(neg_hbm.at[pl.ds(r0, CHUNK)],
                              nb.at[dst], sem).wait()

    def compute(g, slot, colc0):
        def row_body(r, carry):
            srow = slot * CHUNK + r
            rsplat = jnp.full((16,), srow, jnp.int32)
            q2 = jnp.zeros((16,), jnp.float32)
            dpp = jnp.zeros((16,), jnp.float32)
            dpn = jnp.zeros((16,), jnp.float32)
            n2p = jnp.zeros((16,), jnp.float32)
            n2n = jnp.zeros((16,), jnp.float32)
            for j in range(JV):
                qv = qb[srow, pl.ds(16 * j, 16)]
                q2 = q2 + qv * qv
                cidx = (colc0 + 2 * j) + hi
                cp = plsc.load_gather(pb, [rsplat, cidx])
                cn = plsc.load_gather(nb, [rsplat, cidx])
                toff = tpat + (2 * j) * MW
                tp = plsc.load_gather(tab_v, [cp + toff])
                tn = plsc.load_gather(tab_v, [cn + toff])
                dpp = dpp + qv * tp
                n2p = n2p + tp * tp
                dpn = dpn + qv * tn
                n2n = n2n + tn * tn
            col = jnp.full((16,), lax.rem(g, 8) * CHUNK + r, jnp.int32)
            accs = (q2, dpp, dpn, n2p, n2n)
            for i in range(NACC):
                plsc.store_scatter(ob, [i * 16 + iota, col], accs[i])
            return carry
        lax.fori_loop(0, CHUNK, row_body, 0)

    def half(colq0, colc0):
        fire(0, 0, sem0, colq0)
        fire(1, 1, sem1, colq0)

        def outer(g2, carry):
            g = 2 * g2
            wait(g, 0, sem0, colq0)
            compute(g, 0, colc0)

            @pl.when(g2 < NCHUNK // 2 - 1)
            def _():
                fire(g + 2, 0, sem0, colq0)

            wait(g + 1, 1, sem1, colq0)
            compute(g + 1, 1, colc0)

            @pl.when(g2 < NCHUNK // 2 - 1)
            def _():
                fire(g + 3, 1, sem1, colq0)

            @pl.when(lax.rem(g2, 4) == 3)
            def _():
                col0 = base_row + (g2 // 4) * 128
                pltpu.sync_copy(ob, out_hbm.at[c, :, pl.ds(col0, 128)])

            return carry

        lax.fori_loop(0, NCHUNK // 2, outer, 0)

    @pl.when(c == 0)
    def _():
        half(0, 0)

    @pl.when(c == 1)
    def _():
        half(DH, MH)


_sc_partials = functools.partial(
    pl.kernel,
    out_type=jax.ShapeDtypeStruct((HALVES, NACC * 16, B), jnp.float32),
    mesh=plsc.VectorSubcoreMesh(core_axis_name="c", subcore_axis_name="s"),
    compiler_params=pltpu.CompilerParams(
        use_tc_tiling_on_sc=True, needs_layout_passes=False),
    scratch_types=[
        pltpu.VMEM((TW,), jnp.float32),
        pltpu.VMEM((2 * CHUNK, DH), jnp.float32),
        pltpu.VMEM((2 * CHUNK, M), jnp.int32),
        pltpu.VMEM((2 * CHUNK, M), jnp.int32),
        pltpu.VMEM((NACC * 16, 128), jnp.float32),
        pltpu.SemaphoreType.DMA,
        pltpu.SemaphoreType.DMA,
    ],
)(_sc_body)


def _loss_body(x_ref, o_ref):
    x = x_ref[...]
    y = x[0] + x[1]

    def seg(i):
        return jnp.sum(y[16 * i:16 * (i + 1), :], axis=0, keepdims=True)

    q2, dpp, dpn, n2p, n2n = seg(0), seg(1), seg(2), seg(3), seg(4)
    eps = 1e-8
    nq = jnp.maximum(jnp.sqrt(q2), eps)
    sp = dpp / (nq * jnp.maximum(jnp.sqrt(n2p), eps))
    sn = dpn / (nq * jnp.maximum(jnp.sqrt(n2n), eps))
    mx = jnp.maximum(sp, sn)
    lse = jnp.log(jnp.exp(sp - mx) + jnp.exp(sn - mx)) + mx
    o_ref[0, 0] = jnp.sum(lse - sp) * (1.0 / B)


def kernel(q, pos_codes, neg_codes, codebooks):
    tab = codebooks.transpose(0, 2, 1).reshape(HALVES, TW)
    x = _sc_partials(q, pos_codes, neg_codes, tab)
    loss = pl.pallas_call(
        _loss_body,
        out_specs=pl.BlockSpec(memory_space=pltpu.SMEM),
        out_shape=jax.ShapeDtypeStruct((1, 1), jnp.float32),
    )(x)
    return loss[0, 0]


# speedup vs baseline: 1.0959x; 1.0959x over previous
import functools

import jax
import jax.numpy as jnp
from jax import lax
from jax.experimental import pallas as pl
from jax.experimental.pallas import tpu as pltpu
from jax.experimental.pallas import tpu_sc as plsc

B, M, K, DSUB = 16384, 96, 256, 8
D = M * DSUB
HALVES = 2
MH = M // HALVES
DH = D // HALVES
TW = MH * K * DSUB
MW = K * DSUB
NSUB = 16
ROWS_PER_SUB = B // NSUB
CHUNK = 16
NCHUNK = ROWS_PER_SUB // CHUNK
NACC = 5
OW = NACC * 16
JV = DH // 16


def _sc_body(q_hbm, pos_hbm, neg_hbm, tab_hbm, out_hbm,
             tab_v, qb, pb, nb, ob, sem0, sem1):
    c = lax.axis_index("c")
    s = lax.axis_index("s")
    base_row = s * ROWS_PER_SUB

    pltpu.sync_copy(tab_hbm.at[c], tab_v)

    iota = lax.broadcasted_iota(jnp.int32, (16,), 0)
    hi = lax.shift_right_logical(iota, 3)
    d8 = lax.bitwise_and(iota, 7)
    tpat = hi * MW + d8 * K

    def fire(g, slot, sem, colq0):
        r0 = base_row + g * CHUNK
        dst = pl.ds(slot * CHUNK, CHUNK)
        pltpu.async_copy(q_hbm.at[pl.ds(r0, CHUNK), pl.ds(colq0, DH)],
                         qb.at[dst], sem)
        pltpu.async_copy(pos_hbm.at[pl.ds(r0, CHUNK)], pb.at[dst], sem)
        pltpu.async_copy(neg_hbm.at[pl.ds(r0, CHUNK)], nb.at[dst], sem)

    def wait(g, slot, sem, colq0):
        r0 = base_row + g * CHUNK
        dst = pl.ds(slot * CHUNK, CHUNK)
        pltpu.make_async_copy(
            q_hbm.at[pl.ds(r0, CHUNK), pl.ds(colq0, DH)],
            qb.at[dst], sem).wait()
        pltpu.make_async_copy(pos_hbm.at[pl.ds(r0, CHUNK)],
                              pb.at[dst], sem).wait()
        pltpu.make_async_copy(neg_hbm.at[pl.ds(r0, CHUNK)],
                              nb.at[dst], sem).wait()

    def compute(g, slot, colc0):
        def row_body(r, carry):
            srow = slot * CHUNK + r
            rsplat = jnp.full((16,), srow, jnp.int32)
            q2 = jnp.zeros((16,), jnp.float32)
            dpp = jnp.zeros((16,), jnp.float32)
            dpn = jnp.zeros((16,), jnp.float32)
            n2p = jnp.zeros((16,), jnp.float32)
            n2n = jnp.zeros((16,), jnp.float32)
            for j in range(JV):
                qv = qb[srow, pl.ds(16 * j, 16)]
                q2 = q2 + qv * qv
                cidx = (colc0 + 2 * j) + hi
                cp = plsc.load_gather(pb, [rsplat, cidx])
                cn = plsc.load_gather(nb, [rsplat, cidx])
                toff = tpat + (2 * j) * MW
                tp = plsc.load_gather(tab_v, [cp + toff])
                tn = plsc.load_gather(tab_v, [cn + toff])
                dpp = dpp + qv * tp
                n2p = n2p + tp * tp
                dpn = dpn + qv * tn
                n2n = n2n + tn * tn
            ob[srow, pl.ds(0, 16)] = q2
            ob[srow, pl.ds(16, 16)] = dpp
            ob[srow, pl.ds(32, 16)] = dpn
            ob[srow, pl.ds(48, 16)] = n2p
            ob[srow, pl.ds(64, 16)] = n2n
            return carry
        lax.fori_loop(0, CHUNK, row_body, 0)
        pltpu.sync_copy(
            ob.at[pl.ds(slot * CHUNK, CHUNK)],
            out_hbm.at[c, pl.ds(base_row + g * CHUNK, CHUNK)])

    def half(colq0, colc0):
        fire(0, 0, sem0, colq0)
        fire(1, 1, sem1, colq0)

        def outer(g2, carry):
            g = 2 * g2
            wait(g, 0, sem0, colq0)
            compute(g, 0, colc0)

            @pl.when(g2 < NCHUNK // 2 - 1)
            def _():
                fire(g + 2, 0, sem0, colq0)

            wait(g + 1, 1, sem1, colq0)
            compute(g + 1, 1, colc0)

            @pl.when(g2 < NCHUNK // 2 - 1)
            def _():
                fire(g + 3, 1, sem1, colq0)

            return carry

        lax.fori_loop(0, NCHUNK // 2, outer, 0)

    @pl.when(c == 0)
    def _():
        half(0, 0)

    @pl.when(c == 1)
    def _():
        half(DH, MH)


_sc_partials = functools.partial(
    pl.kernel,
    out_type=jax.ShapeDtypeStruct((HALVES, B, OW), jnp.float32),
    mesh=plsc.VectorSubcoreMesh(core_axis_name="c", subcore_axis_name="s"),
    compiler_params=pltpu.CompilerParams(
        use_tc_tiling_on_sc=True, needs_layout_passes=False),
    scratch_types=[
        pltpu.VMEM((TW,), jnp.float32),
        pltpu.VMEM((2 * CHUNK, DH), jnp.float32),
        pltpu.VMEM((2 * CHUNK, M), jnp.int32),
        pltpu.VMEM((2 * CHUNK, M), jnp.int32),
        pltpu.VMEM((2 * CHUNK, OW), jnp.float32),
        pltpu.SemaphoreType.DMA,
        pltpu.SemaphoreType.DMA,
    ],
)(_sc_body)


def _loss_body(x_ref, o_ref):
    x = x_ref[...]
    y = x[0] + x[1]
    i80 = lax.broadcasted_iota(jnp.int32, (OW, NACC), 0)
    i5 = lax.broadcasted_iota(jnp.int32, (OW, NACC), 1)
    sel = jnp.where(i80 // 16 == i5, 1.0, 0.0).astype(jnp.float32)
    z = jnp.dot(y, sel, preferred_element_type=jnp.float32)
    q2 = z[:, 0:1]
    dpp = z[:, 1:2]
    dpn = z[:, 2:3]
    n2p = z[:, 3:4]
    n2n = z[:, 4:5]
    eps = 1e-8
    nq = jnp.maximum(jnp.sqrt(q2), eps)
    sp = dpp / (nq * jnp.maximum(jnp.sqrt(n2p), eps))
    sn = dpn / (nq * jnp.maximum(jnp.sqrt(n2n), eps))
    mx = jnp.maximum(sp, sn)
    lse = jnp.log(jnp.exp(sp - mx) + jnp.exp(sn - mx)) + mx
    o_ref[0, 0] = jnp.sum(lse - sp) * (1.0 / B)


def kernel(q, pos_codes, neg_codes, codebooks):
    tab = codebooks.transpose(0, 2, 1).reshape(HALVES, TW)
    x = _sc_partials(q, pos_codes, neg_codes, tab)
    loss = pl.pallas_call(
        _loss_body,
        out_specs=pl.BlockSpec(memory_space=pltpu.SMEM),
        out_shape=jax.ShapeDtypeStruct((1, 1), jnp.float32),
    )(x)
    return loss[0, 0]


# speedup vs baseline: 1.6545x; 1.5096x over previous
import functools

import jax
import jax.numpy as jnp
from jax import lax
from jax.experimental import pallas as pl
from jax.experimental.pallas import tpu as pltpu
from jax.experimental.pallas import tpu_sc as plsc

B, M, K, DSUB = 16384, 96, 256, 8
D = M * DSUB
HALVES = 2
MH = M // HALVES
DH = D // HALVES
TW = MH * K * DSUB
MW = K * DSUB
NSUB = 16
ROWS_PER_SUB = B // NSUB
CHUNK = 16
NCHUNK = ROWS_PER_SUB // CHUNK
NACC = 5
OW = NACC * 16
JV = DH // 16


def _sc_body(q_hbm, pos_hbm, neg_hbm, tab_hbm, out_hbm,
             tab_v, qb, pb, nb, ob, sem0, sem1):
    c = lax.axis_index("c")
    s = lax.axis_index("s")
    base_row = s * ROWS_PER_SUB

    pltpu.sync_copy(tab_hbm.at[c], tab_v)

    iota = lax.broadcasted_iota(jnp.int32, (16,), 0)
    hi = lax.shift_right_logical(iota, 3)
    d8 = lax.bitwise_and(iota, 7)
    tpat = hi * MW + d8

    def fire(g, slot, sem, colq0):
        r0 = base_row + g * CHUNK
        dst = pl.ds(slot * CHUNK, CHUNK)
        pltpu.async_copy(q_hbm.at[pl.ds(r0, CHUNK), pl.ds(colq0, DH)],
                         qb.at[dst], sem)
        pltpu.async_copy(pos_hbm.at[pl.ds(r0, CHUNK)], pb.at[dst], sem)
        pltpu.async_copy(neg_hbm.at[pl.ds(r0, CHUNK)], nb.at[dst], sem)

    def wait(g, slot, sem, colq0):
        r0 = base_row + g * CHUNK
        dst = pl.ds(slot * CHUNK, CHUNK)
        pltpu.make_async_copy(
            q_hbm.at[pl.ds(r0, CHUNK), pl.ds(colq0, DH)],
            qb.at[dst], sem).wait()
        pltpu.make_async_copy(pos_hbm.at[pl.ds(r0, CHUNK)],
                              pb.at[dst], sem).wait()
        pltpu.make_async_copy(neg_hbm.at[pl.ds(r0, CHUNK)],
                              nb.at[dst], sem).wait()

    def compute(g, slot, colc0):
        def row_body(r, carry):
            srow = slot * CHUNK + r
            rsplat = jnp.full((16,), srow, jnp.int32)
            q2 = jnp.zeros((16,), jnp.float32)
            dpp = jnp.zeros((16,), jnp.float32)
            dpn = jnp.zeros((16,), jnp.float32)
            n2p = jnp.zeros((16,), jnp.float32)
            n2n = jnp.zeros((16,), jnp.float32)
            for j in range(JV):
                qv = qb[srow, pl.ds(16 * j, 16)]
                q2 = q2 + qv * qv
                cidx = (colc0 + 2 * j) + hi
                cp = plsc.load_gather(pb, [rsplat, cidx])
                cn = plsc.load_gather(nb, [rsplat, cidx])
                toff = tpat + (2 * j) * MW
                tp = plsc.load_gather(tab_v, [cp * DSUB + toff])
                tn = plsc.load_gather(tab_v, [cn * DSUB + toff])
                dpp = dpp + qv * tp
                n2p = n2p + tp * tp
                dpn = dpn + qv * tn
                n2n = n2n + tn * tn
            ob[srow, pl.ds(0, 16)] = q2
            ob[srow, pl.ds(16, 16)] = dpp
            ob[srow, pl.ds(32, 16)] = dpn
            ob[srow, pl.ds(48, 16)] = n2p
            ob[srow, pl.ds(64, 16)] = n2n
            return carry
        lax.fori_loop(0, CHUNK, row_body, 0)
        pltpu.sync_copy(
            ob.at[pl.ds(slot * CHUNK, CHUNK)],
            out_hbm.at[c, pl.ds(base_row + g * CHUNK, CHUNK)])

    def half(colq0, colc0):
        fire(0, 0, sem0, colq0)
        fire(1, 1, sem1, colq0)

        def outer(g2, carry):
            g = 2 * g2
            wait(g, 0, sem0, colq0)
            compute(g, 0, colc0)

            @pl.when(g2 < NCHUNK // 2 - 1)
            def _():
                fire(g + 2, 0, sem0, colq0)

            wait(g + 1, 1, sem1, colq0)
            compute(g + 1, 1, colc0)

            @pl.when(g2 < NCHUNK // 2 - 1)
            def _():
                fire(g + 3, 1, sem1, colq0)

            return carry

        lax.fori_loop(0, NCHUNK // 2, outer, 0)

    @pl.when(c == 0)
    def _():
        half(0, 0)

    @pl.when(c == 1)
    def _():
        half(DH, MH)


_sc_partials = functools.partial(
    pl.kernel,
    out_type=jax.ShapeDtypeStruct((HALVES, B, OW), jnp.float32),
    mesh=plsc.VectorSubcoreMesh(core_axis_name="c", subcore_axis_name="s"),
    compiler_params=pltpu.CompilerParams(
        use_tc_tiling_on_sc=True, needs_layout_passes=False),
    scratch_types=[
        pltpu.VMEM((TW,), jnp.float32),
        pltpu.VMEM((2 * CHUNK, DH), jnp.float32),
        pltpu.VMEM((2 * CHUNK, M), jnp.int32),
        pltpu.VMEM((2 * CHUNK, M), jnp.int32),
        pltpu.VMEM((2 * CHUNK, OW), jnp.float32),
        pltpu.SemaphoreType.DMA,
        pltpu.SemaphoreType.DMA,
    ],
)(_sc_body)


def _loss_body(x_ref, o_ref):
    x = x_ref[...]
    y = x[0] + x[1]
    i80 = lax.broadcasted_iota(jnp.int32, (OW, NACC), 0)
    i5 = lax.broadcasted_iota(jnp.int32, (OW, NACC), 1)
    sel = jnp.where(i80 // 16 == i5, 1.0, 0.0).astype(jnp.float32)
    z = jnp.dot(y, sel, preferred_element_type=jnp.float32)
    q2 = z[:, 0:1]
    dpp = z[:, 1:2]
    dpn = z[:, 2:3]
    n2p = z[:, 3:4]
    n2n = z[:, 4:5]
    eps = 1e-8
    nq = jnp.maximum(jnp.sqrt(q2), eps)
    sp = dpp / (nq * jnp.maximum(jnp.sqrt(n2p), eps))
    sn = dpn / (nq * jnp.maximum(jnp.sqrt(n2n), eps))
    mx = jnp.maximum(sp, sn)
    lse = jnp.log(jnp.exp(sp - mx) + jnp.exp(sn - mx)) + mx
    o_ref[0, 0] = jnp.sum(lse - sp) * (1.0 / B)


def kernel(q, pos_codes, neg_codes, codebooks):
    tab = codebooks.reshape(HALVES, TW)
    x = _sc_partials(q, pos_codes, neg_codes, tab)
    loss = pl.pallas_call(
        _loss_body,
        out_specs=pl.BlockSpec(memory_space=pltpu.SMEM),
        out_shape=jax.ShapeDtypeStruct((1, 1), jnp.float32),
    )(x)
    return loss[0, 0]


# speedup vs baseline: 1.7803x; 1.0761x over previous
import functools

import jax
import jax.numpy as jnp
from jax import lax
from jax.experimental import pallas as pl
from jax.experimental.pallas import tpu as pltpu
from jax.experimental.pallas import tpu_sc as plsc

B, M, K, DSUB = 16384, 96, 256, 8
D = M * DSUB
HALVES = 2
MH = M // HALVES
DH = D // HALVES
TW = MH * K * DSUB
MW = K * DSUB
NSUB = 16
ROWS_PER_SUB = B // NSUB
CHUNK = 16
NCHUNK = ROWS_PER_SUB // CHUNK
NACC = 5
OW = NACC * 16
JV = DH // 16


def _sc_body(q_hbm, pos_hbm, neg_hbm, tab_hbm, out_hbm,
             tab_v, qb, pb, nb, ob, sem0, sem1):
    c = lax.axis_index("c")
    s = lax.axis_index("s")
    base_row = s * ROWS_PER_SUB

    pltpu.sync_copy(tab_hbm.at[c], tab_v)

    iota = lax.broadcasted_iota(jnp.int32, (16,), 0)
    hi = lax.shift_right_logical(iota, 3)
    d8 = lax.bitwise_and(iota, 7)
    tpat = hi * MW + d8

    def fire(g, slot, sem, colq0):
        r0 = base_row + g * CHUNK
        dst = pl.ds(slot * CHUNK, CHUNK)
        pltpu.async_copy(q_hbm.at[pl.ds(r0, CHUNK), pl.ds(colq0, DH)],
                         qb.at[dst], sem)
        pltpu.async_copy(pos_hbm.at[pl.ds(r0, CHUNK)], pb.at[dst], sem)
        pltpu.async_copy(neg_hbm.at[pl.ds(r0, CHUNK)], nb.at[dst], sem)

    def wait(g, slot, sem, colq0):
        r0 = base_row + g * CHUNK
        dst = pl.ds(slot * CHUNK, CHUNK)
        pltpu.make_async_copy(
            q_hbm.at[pl.ds(r0, CHUNK), pl.ds(colq0, DH)],
            qb.at[dst], sem).wait()
        pltpu.make_async_copy(pos_hbm.at[pl.ds(r0, CHUNK)],
                              pb.at[dst], sem).wait()
        pltpu.make_async_copy(neg_hbm.at[pl.ds(r0, CHUNK)],
                              nb.at[dst], sem).wait()

    def compute(g, slot, colc0):
        def row_body(r, carry):
            srow = slot * CHUNK + r
            cvp = [pb[srow, pl.ds(colc0 + 16 * k, 16)] * DSUB for k in range(3)]
            cvn = [nb[srow, pl.ds(colc0 + 16 * k, 16)] * DSUB for k in range(3)]
            q2 = jnp.zeros((16,), jnp.float32)
            dpp = jnp.zeros((16,), jnp.float32)
            dpn = jnp.zeros((16,), jnp.float32)
            n2p = jnp.zeros((16,), jnp.float32)
            n2n = jnp.zeros((16,), jnp.float32)
            for j in range(JV):
                qv = qb[srow, pl.ds(16 * j, 16)]
                q2 = q2 + qv * qv
                pat = hi + (2 * j) % 16
                cp = jnp.take_along_axis(cvp[j // 8], pat, axis=0,
                                         mode="promise_in_bounds")
                cn = jnp.take_along_axis(cvn[j // 8], pat, axis=0,
                                         mode="promise_in_bounds")
                toff = tpat + (2 * j) * MW
                tp = plsc.load_gather(tab_v, [cp + toff])
                tn = plsc.load_gather(tab_v, [cn + toff])
                dpp = dpp + qv * tp
                n2p = n2p + tp * tp
                dpn = dpn + qv * tn
                n2n = n2n + tn * tn
            ob[srow, pl.ds(0, 16)] = q2
            ob[srow, pl.ds(16, 16)] = dpp
            ob[srow, pl.ds(32, 16)] = dpn
            ob[srow, pl.ds(48, 16)] = n2p
            ob[srow, pl.ds(64, 16)] = n2n
            return carry
        lax.fori_loop(0, CHUNK, row_body, 0)
        pltpu.sync_copy(
            ob.at[pl.ds(slot * CHUNK, CHUNK)],
            out_hbm.at[c, pl.ds(base_row + g * CHUNK, CHUNK)])

    def half(colq0, colc0):
        fire(0, 0, sem0, colq0)
        fire(1, 1, sem1, colq0)

        def outer(g2, carry):
            g = 2 * g2
            wait(g, 0, sem0, colq0)
            compute(g, 0, colc0)

            @pl.when(g2 < NCHUNK // 2 - 1)
            def _():
                fire(g + 2, 0, sem0, colq0)

            wait(g + 1, 1, sem1, colq0)
            compute(g + 1, 1, colc0)

            @pl.when(g2 < NCHUNK // 2 - 1)
            def _():
                fire(g + 3, 1, sem1, colq0)

            return carry

        lax.fori_loop(0, NCHUNK // 2, outer, 0)

    @pl.when(c == 0)
    def _():
        half(0, 0)

    @pl.when(c == 1)
    def _():
        half(DH, MH)


_sc_partials = functools.partial(
    pl.kernel,
    out_type=jax.ShapeDtypeStruct((HALVES, B, OW), jnp.float32),
    mesh=plsc.VectorSubcoreMesh(core_axis_name="c", subcore_axis_name="s"),
    compiler_params=pltpu.CompilerParams(
        use_tc_tiling_on_sc=True, needs_layout_passes=False),
    scratch_types=[
        pltpu.VMEM((TW,), jnp.float32),
        pltpu.VMEM((2 * CHUNK, DH), jnp.float32),
        pltpu.VMEM((2 * CHUNK, M), jnp.int32),
        pltpu.VMEM((2 * CHUNK, M), jnp.int32),
        pltpu.VMEM((2 * CHUNK, OW), jnp.float32),
        pltpu.SemaphoreType.DMA,
        pltpu.SemaphoreType.DMA,
    ],
)(_sc_body)


def _loss_body(x_ref, o_ref):
    x = x_ref[...]
    y = x[0] + x[1]
    i80 = lax.broadcasted_iota(jnp.int32, (OW, NACC), 0)
    i5 = lax.broadcasted_iota(jnp.int32, (OW, NACC), 1)
    sel = jnp.where(i80 // 16 == i5, 1.0, 0.0).astype(jnp.float32)
    z = jnp.dot(y, sel, preferred_element_type=jnp.float32)
    q2 = z[:, 0:1]
    dpp = z[:, 1:2]
    dpn = z[:, 2:3]
    n2p = z[:, 3:4]
    n2n = z[:, 4:5]
    eps = 1e-8
    nq = jnp.maximum(jnp.sqrt(q2), eps)
    sp = dpp / (nq * jnp.maximum(jnp.sqrt(n2p), eps))
    sn = dpn / (nq * jnp.maximum(jnp.sqrt(n2n), eps))
    mx = jnp.maximum(sp, sn)
    lse = jnp.log(jnp.exp(sp - mx) + jnp.exp(sn - mx)) + mx
    o_ref[0, 0] = jnp.sum(lse - sp) * (1.0 / B)


def kernel(q, pos_codes, neg_codes, codebooks):
    tab = codebooks.reshape(HALVES, TW)
    x = _sc_partials(q, pos_codes, neg_codes, tab)
    loss = pl.pallas_call(
        _loss_body,
        out_specs=pl.BlockSpec(memory_space=pltpu.SMEM),
        out_shape=jax.ShapeDtypeStruct((1, 1), jnp.float32),
    )(x)
    return loss[0, 0]


# speedup vs baseline: 1.7977x; 1.0098x over previous
import functools

import jax
import jax.numpy as jnp
from jax import lax
from jax.experimental import pallas as pl
from jax.experimental.pallas import tpu as pltpu
from jax.experimental.pallas import tpu_sc as plsc

B, M, K, DSUB = 16384, 96, 256, 8
D = M * DSUB
HALVES = 2
MH = M // HALVES
DH = D // HALVES
TW = MH * K * DSUB
MW = K * DSUB
NSUB = 16
ROWS_PER_SUB = B // NSUB
CHUNK = 16
NCHUNK = ROWS_PER_SUB // CHUNK
NACC = 5
OW = NACC * 16
JV = DH // 16


def _sc_body(q_hbm, pos_hbm, neg_hbm, tab_hbm, out_hbm,
             tab_v, qb, pb, nb, ob, sem0, sem1):
    c = lax.axis_index("c")
    s = lax.axis_index("s")
    base_row = s * ROWS_PER_SUB

    pltpu.sync_copy(tab_hbm.at[c], tab_v)

    iota = lax.broadcasted_iota(jnp.int32, (16,), 0)
    hi = lax.shift_right_logical(iota, 3)
    d8 = lax.bitwise_and(iota, 7)
    tpat = hi * MW + d8

    def fire(g, slot, sem, colq0):
        r0 = base_row + g * CHUNK
        dst = pl.ds(slot * CHUNK, CHUNK)
        pltpu.async_copy(q_hbm.at[pl.ds(r0, CHUNK), pl.ds(colq0, DH)],
                         qb.at[dst], sem)
        pltpu.async_copy(pos_hbm.at[pl.ds(r0, CHUNK)], pb.at[dst], sem)
        pltpu.async_copy(neg_hbm.at[pl.ds(r0, CHUNK)], nb.at[dst], sem)

    def wait(g, slot, sem, colq0):
        r0 = base_row + g * CHUNK
        dst = pl.ds(slot * CHUNK, CHUNK)
        pltpu.make_async_copy(
            q_hbm.at[pl.ds(r0, CHUNK), pl.ds(colq0, DH)],
            qb.at[dst], sem).wait()
        pltpu.make_async_copy(pos_hbm.at[pl.ds(r0, CHUNK)],
                              pb.at[dst], sem).wait()
        pltpu.make_async_copy(neg_hbm.at[pl.ds(r0, CHUNK)],
                              nb.at[dst], sem).wait()

    def compute(g, slot, colc0):
        def row_body(r, carry):
            srow = slot * CHUNK + r
            cvp = [pb[srow, pl.ds(colc0 + 16 * k, 16)] * DSUB for k in range(3)]
            cvn = [nb[srow, pl.ds(colc0 + 16 * k, 16)] * DSUB for k in range(3)]
            q2 = jnp.zeros((16,), jnp.float32)
            dpp = jnp.zeros((16,), jnp.float32)
            dpn = jnp.zeros((16,), jnp.float32)
            n2p = jnp.zeros((16,), jnp.float32)
            n2n = jnp.zeros((16,), jnp.float32)
            for j in range(JV):
                qv = qb[srow, pl.ds(16 * j, 16)]
                q2 = q2 + qv * qv
                pat = hi + (2 * j) % 16
                cp = jnp.take_along_axis(cvp[j // 8], pat, axis=0,
                                         mode="promise_in_bounds")
                cn = jnp.take_along_axis(cvn[j // 8], pat, axis=0,
                                         mode="promise_in_bounds")
                toff = tpat + (2 * j) * MW
                tp = plsc.load_gather(tab_v, [cp + toff])
                tn = plsc.load_gather(tab_v, [cn + toff])
                dpp = dpp + qv * tp
                n2p = n2p + tp * tp
                dpn = dpn + qv * tn
                n2n = n2n + tn * tn
            ob[srow, pl.ds(0, 16)] = q2
            ob[srow, pl.ds(16, 16)] = dpp
            ob[srow, pl.ds(32, 16)] = dpn
            ob[srow, pl.ds(48, 16)] = n2p
            ob[srow, pl.ds(64, 16)] = n2n
            return carry
        lax.fori_loop(0, CHUNK, row_body, 0)
        pltpu.sync_copy(
            ob.at[pl.ds(slot * CHUNK, CHUNK)],
            out_hbm.at[c, pl.ds(base_row + g * CHUNK, CHUNK)])

    def half(colq0, colc0):
        fire(0, 0, sem0, colq0)
        fire(1, 1, sem1, colq0)

        def outer(g2, carry):
            g = 2 * g2
            wait(g, 0, sem0, colq0)
            compute(g, 0, colc0)

            @pl.when(g2 < NCHUNK // 2 - 1)
            def _():
                fire(g + 2, 0, sem0, colq0)

            wait(g + 1, 1, sem1, colq0)
            compute(g + 1, 1, colc0)

            @pl.when(g2 < NCHUNK // 2 - 1)
            def _():
                fire(g + 3, 1, sem1, colq0)

            return carry

        lax.fori_loop(0, NCHUNK // 2, outer, 0)

    @pl.when(c == 0)
    def _():
        half(0, 0)

    @pl.when(c == 1)
    def _():
        half(DH, MH)


_sc_partials = functools.partial(
    pl.kernel,
    out_type=jax.ShapeDtypeStruct((HALVES, B, OW), jnp.float32),
    mesh=plsc.VectorSubcoreMesh(core_axis_name="c", subcore_axis_name="s"),
    compiler_params=pltpu.CompilerParams(
        use_tc_tiling_on_sc=True, needs_layout_passes=False),
    scratch_types=[
        pltpu.VMEM((TW,), jnp.float32),
        pltpu.VMEM((2 * CHUNK, DH), jnp.float32),
        pltpu.VMEM((2 * CHUNK, M), jnp.int32),
        pltpu.VMEM((2 * CHUNK, M), jnp.int32),
        pltpu.VMEM((2 * CHUNK, OW), jnp.float32),
        pltpu.SemaphoreType.DMA,
        pltpu.SemaphoreType.DMA,
    ],
)(_sc_body)


BB = 2048


def _loss_body(x_ref, o_ref):
    x = x_ref[...]
    y = x[0] + x[1]
    i80 = lax.broadcasted_iota(jnp.int32, (OW, NACC), 0)
    i5 = lax.broadcasted_iota(jnp.int32, (OW, NACC), 1)
    sel = jnp.where(i80 // 16 == i5, 1.0, 0.0).astype(jnp.float32)
    z = jnp.dot(y, sel, preferred_element_type=jnp.float32)
    q2 = z[:, 0:1]
    dpp = z[:, 1:2]
    dpn = z[:, 2:3]
    n2p = z[:, 3:4]
    n2n = z[:, 4:5]
    eps = 1e-8
    nq = jnp.maximum(jnp.sqrt(q2), eps)
    sp = dpp / (nq * jnp.maximum(jnp.sqrt(n2p), eps))
    sn = dpn / (nq * jnp.maximum(jnp.sqrt(n2n), eps))
    mx = jnp.maximum(sp, sn)
    lse = jnp.log(jnp.exp(sp - mx) + jnp.exp(sn - mx)) + mx
    part = jnp.sum(lse - sp) * (1.0 / B)

    @pl.when(pl.program_id(0) == 0)
    def _():
        o_ref[0, 0] = 0.0

    o_ref[0, 0] += part


def kernel(q, pos_codes, neg_codes, codebooks):
    tab = codebooks.reshape(HALVES, TW)
    x = _sc_partials(q, pos_codes, neg_codes, tab)
    loss = pl.pallas_call(
        _loss_body,
        grid=(B // BB,),
        in_specs=[pl.BlockSpec((HALVES, BB, OW), lambda i: (0, i, 0))],
        out_specs=pl.BlockSpec(memory_space=pltpu.SMEM),
        out_shape=jax.ShapeDtypeStruct((1, 1), jnp.float32),
    )(x)
    return loss[0, 0]


# speedup vs baseline: 1.7978x; 1.0001x over previous
import functools

import jax
import jax.numpy as jnp
from jax import lax
from jax.experimental import pallas as pl
from jax.experimental.pallas import tpu as pltpu
from jax.experimental.pallas import tpu_sc as plsc

B, M, K, DSUB = 16384, 96, 256, 8
D = M * DSUB
HALVES = 2
MH = M // HALVES
DH = D // HALVES
TW = MH * K * DSUB
MW = K * DSUB
NSUB = 16
ROWS_PER_SUB = B // NSUB
CHUNK = 16
NCHUNK = ROWS_PER_SUB // CHUNK
NACC = 5
OW = NACC * 16
JV = DH // 16


def _sc_body(q_hbm, pos_hbm, neg_hbm, tab_hbm, out_hbm,
             tab_v, qb, pb, nb, ob, sem0, sem1):
    c = lax.axis_index("c")
    s = lax.axis_index("s")
    base_row = s * ROWS_PER_SUB

    pltpu.sync_copy(tab_hbm.at[c], tab_v)

    iota = lax.broadcasted_iota(jnp.int32, (16,), 0)
    hi = lax.shift_right_logical(iota, 3)
    d8 = lax.bitwise_and(iota, 7)
    tpat = hi * MW + d8

    def fire(g, slot, sem, colq0):
        r0 = base_row + g * CHUNK
        dst = pl.ds(slot * CHUNK, CHUNK)
        pltpu.async_copy(q_hbm.at[pl.ds(r0, CHUNK), pl.ds(colq0, DH)],
                         qb.at[dst], sem)
        pltpu.async_copy(pos_hbm.at[pl.ds(r0, CHUNK)], pb.at[dst], sem)
        pltpu.async_copy(neg_hbm.at[pl.ds(r0, CHUNK)], nb.at[dst], sem)

    def wait(g, slot, sem, colq0):
        r0 = base_row + g * CHUNK
        dst = pl.ds(slot * CHUNK, CHUNK)
        pltpu.make_async_copy(
            q_hbm.at[pl.ds(r0, CHUNK), pl.ds(colq0, DH)],
            qb.at[dst], sem).wait()
        pltpu.make_async_copy(pos_hbm.at[pl.ds(r0, CHUNK)],
                              pb.at[dst], sem).wait()
        pltpu.make_async_copy(neg_hbm.at[pl.ds(r0, CHUNK)],
                              nb.at[dst], sem).wait()

    def compute(g, slot, colc0):
        def row_one(srow):
            cvp = [pb[srow, pl.ds(colc0 + 16 * k, 16)] * DSUB for k in range(3)]
            cvn = [nb[srow, pl.ds(colc0 + 16 * k, 16)] * DSUB for k in range(3)]
            q2 = jnp.zeros((16,), jnp.float32)
            dpp = jnp.zeros((16,), jnp.float32)
            dpn = jnp.zeros((16,), jnp.float32)
            n2p = jnp.zeros((16,), jnp.float32)
            n2n = jnp.zeros((16,), jnp.float32)
            for j in range(JV):
                qv = qb[srow, pl.ds(16 * j, 16)]
                q2 = q2 + qv * qv
                pat = hi + (2 * j) % 16
                cp = jnp.take_along_axis(cvp[j // 8], pat, axis=0,
                                         mode="promise_in_bounds")
                cn = jnp.take_along_axis(cvn[j // 8], pat, axis=0,
                                         mode="promise_in_bounds")
                toff = tpat + (2 * j) * MW
                tp = plsc.load_gather(tab_v, [cp + toff])
                tn = plsc.load_gather(tab_v, [cn + toff])
                dpp = dpp + qv * tp
                n2p = n2p + tp * tp
                dpn = dpn + qv * tn
                n2n = n2n + tn * tn
            ob[srow, pl.ds(0, 16)] = q2
            ob[srow, pl.ds(16, 16)] = dpp
            ob[srow, pl.ds(32, 16)] = dpn
            ob[srow, pl.ds(48, 16)] = n2p
            ob[srow, pl.ds(64, 16)] = n2n

        def row_body(r2, carry):
            row_one(slot * CHUNK + 2 * r2)
            row_one(slot * CHUNK + 2 * r2 + 1)
            return carry
        lax.fori_loop(0, CHUNK // 2, row_body, 0)
        pltpu.sync_copy(
            ob.at[pl.ds(slot * CHUNK, CHUNK)],
            out_hbm.at[c, pl.ds(base_row + g * CHUNK, CHUNK)])

    def half(colq0, colc0):
        fire(0, 0, sem0, colq0)
        fire(1, 1, sem1, colq0)

        def outer(g2, carry):
            g = 2 * g2
            wait(g, 0, sem0, colq0)
            compute(g, 0, colc0)

            @pl.when(g2 < NCHUNK // 2 - 1)
            def _():
                fire(g + 2, 0, sem0, colq0)

            wait(g + 1, 1, sem1, colq0)
            compute(g + 1, 1, colc0)

            @pl.when(g2 < NCHUNK // 2 - 1)
            def _():
                fire(g + 3, 1, sem1, colq0)

            return carry

        lax.fori_loop(0, NCHUNK // 2, outer, 0)

    @pl.when(c == 0)
    def _():
        half(0, 0)

    @pl.when(c == 1)
    def _():
        half(DH, MH)


_sc_partials = functools.partial(
    pl.kernel,
    out_type=jax.ShapeDtypeStruct((HALVES, B, OW), jnp.float32),
    mesh=plsc.VectorSubcoreMesh(core_axis_name="c", subcore_axis_name="s"),
    compiler_params=pltpu.CompilerParams(
        use_tc_tiling_on_sc=True, needs_layout_passes=False),
    scratch_types=[
        pltpu.VMEM((TW,), jnp.float32),
        pltpu.VMEM((2 * CHUNK, DH), jnp.float32),
        pltpu.VMEM((2 * CHUNK, M), jnp.int32),
        pltpu.VMEM((2 * CHUNK, M), jnp.int32),
        pltpu.VMEM((2 * CHUNK, OW), jnp.float32),
        pltpu.SemaphoreType.DMA,
        pltpu.SemaphoreType.DMA,
    ],
)(_sc_body)


BB = 2048


def _loss_body(x_ref, o_ref):
    x = x_ref[...]
    y = x[0] + x[1]
    i80 = lax.broadcasted_iota(jnp.int32, (OW, NACC), 0)
    i5 = lax.broadcasted_iota(jnp.int32, (OW, NACC), 1)
    sel = jnp.where(i80 // 16 == i5, 1.0, 0.0).astype(jnp.float32)
    z = jnp.dot(y, sel, preferred_element_type=jnp.float32)
    q2 = z[:, 0:1]
    dpp = z[:, 1:2]
    dpn = z[:, 2:3]
    n2p = z[:, 3:4]
    n2n = z[:, 4:5]
    eps = 1e-8
    nq = jnp.maximum(jnp.sqrt(q2), eps)
    sp = dpp / (nq * jnp.maximum(jnp.sqrt(n2p), eps))
    sn = dpn / (nq * jnp.maximum(jnp.sqrt(n2n), eps))
    mx = jnp.maximum(sp, sn)
    lse = jnp.log(jnp.exp(sp - mx) + jnp.exp(sn - mx)) + mx
    part = jnp.sum(lse - sp) * (1.0 / B)

    @pl.when(pl.program_id(0) == 0)
    def _():
        o_ref[0, 0] = 0.0

    o_ref[0, 0] += part


def kernel(q, pos_codes, neg_codes, codebooks):
    tab = codebooks.reshape(HALVES, TW)
    x = _sc_partials(q, pos_codes, neg_codes, tab)
    loss = pl.pallas_call(
        _loss_body,
        grid=(B // BB,),
        in_specs=[pl.BlockSpec((HALVES, BB, OW), lambda i: (0, i, 0))],
        out_specs=pl.BlockSpec(memory_space=pltpu.SMEM),
        out_shape=jax.ShapeDtypeStruct((1, 1), jnp.float32),
    )(x)
    return loss[0, 0]
